# baseline (speedup 1.0000x reference)
import functools

import jax
import jax.numpy as jnp
from jax import lax
from jax.experimental import pallas as pl
from jax.experimental.pallas import tpu as pltpu
from jax.experimental.pallas import tpu_sc as plsc

E = 64
K = 2
D = 1024
H = 1024
T = 8192
CAP = 320
_SC_PARAMS = pltpu.CompilerParams(needs_layout_passes=False)
SLOTS = E * CAP
SLOTS_PAD = 20496
TRASH = SLOTS

NW = 32
TOKW = T // NW
TOKB = 256
NB = T // TOKB


def _gate_body(x_ref, wg_ref, e0_ref, e1_ref, p0_ref, p1_ref, s0_ref, s1_ref,
               stats_ref, laux_ref, acc_ref):
    i = pl.program_id(0)

    @pl.when(i == 0)
    def _init():
        acc_ref[...] = jnp.zeros_like(acc_ref)

    xb = x_ref[...]
    logits = jnp.dot(xb, wg_ref[...], preferred_element_type=jnp.float32)
    m = jnp.max(logits, axis=-1, keepdims=True)
    ex = jnp.exp(logits - m)
    gates = ex / jnp.sum(ex, axis=-1, keepdims=True)

    ii = lax.broadcasted_iota(jnp.int32, (TOKB, E), 1)
    m1 = jnp.max(gates, axis=-1, keepdims=True)
    idx1 = jnp.min(jnp.where(gates == m1, ii, E), axis=-1)
    mask0 = ii == idx1[:, None]
    g2 = jnp.where(mask0, -1.0, gates)
    m2 = jnp.max(g2, axis=-1, keepdims=True)
    idx2 = jnp.min(jnp.where(g2 == m2, ii, E), axis=-1)
    mask1 = ii == idx2[:, None]

    denom = m1[:, 0] + m2[:, 0] + 1e-9
    s0 = m1[:, 0] / denom
    s1 = m2[:, 0] / denom

    mask0f = mask0.astype(jnp.float32)
    mask1f = mask1.astype(jnp.float32)
    r = lax.broadcasted_iota(jnp.int32, (TOKB, TOKB), 0)
    c = lax.broadcasted_iota(jnp.int32, (TOKB, TOKB), 1)
    tril = (r >= c).astype(jnp.float32)
    cum0 = jnp.dot(tril, mask0f, preferred_element_type=jnp.float32)
    cum1 = jnp.dot(tril, mask1f, preferred_element_type=jnp.float32)

    carry0 = acc_ref[0:1, :E]
    carry1 = acc_ref[1:2, :E]
    p0 = jnp.sum(mask0f * (cum0 + carry0 - 1.0), axis=-1)
    p1 = jnp.sum(mask1f * (cum1 + carry1 - 1.0), axis=-1)

    e0_ref[...] = idx1.reshape(1, 1, TOKB)
    e1_ref[...] = idx2.reshape(1, 1, TOKB)
    p0_ref[...] = p0.astype(jnp.int32).reshape(1, 1, TOKB)
    p1_ref[...] = p1.astype(jnp.int32).reshape(1, 1, TOKB)
    s0_ref[...] = s0.reshape(1, 1, TOKB)
    s1_ref[...] = s1.reshape(1, 1, TOKB)

    acc_ref[0:1, :E] = carry0 + jnp.sum(mask0f, axis=0, keepdims=True)
    acc_ref[1:2, :E] = carry1 + jnp.sum(mask1f, axis=0, keepdims=True)
    acc_ref[2:3, :E] = acc_ref[2:3, :E] + jnp.sum(gates, axis=0, keepdims=True)

    stats_ref[...] = acc_ref[...]
    cnt0 = acc_ref[0:1, :E]
    me = acc_ref[2:3, :E]
    laux_ref[0, 0] = (jnp.float32(E) * jnp.sum(me * cnt0)
                      / (jnp.float32(T) * jnp.float32(T)))


def _gate(x, wg):
    i32, f32 = jnp.int32, jnp.float32
    tok3 = lambda dt: jax.ShapeDtypeStruct((NB, 1, TOKB), dt)
    out_shapes = (tok3(i32), tok3(i32), tok3(i32), tok3(i32),
                  tok3(f32), tok3(f32),
                  jax.ShapeDtypeStruct((8, 128), f32),
                  jax.ShapeDtypeStruct((1, 1), f32))
    tokspec = lambda: pl.BlockSpec((1, 1, TOKB), lambda i: (i, 0, 0))
    return pl.pallas_call(
        _gate_body,
        grid=(NB,),
        in_specs=[pl.BlockSpec((TOKB, D), lambda i: (i, 0)),
                  pl.BlockSpec((D, E), lambda i: (0, 0))],
        out_specs=(tokspec(), tokspec(), tokspec(), tokspec(),
                   tokspec(), tokspec(),
                   pl.BlockSpec((8, 128), lambda i: (0, 0)),
                   pl.BlockSpec(memory_space=pltpu.SMEM)),
        out_shape=out_shapes,
        scratch_shapes=[pltpu.VMEM((8, 128), f32)],
    )(x, wg)


def _route_body(e0_hbm, e1_hbm, p0_hbm, p1_hbm, s0_hbm, s1_hbm, cnt_hbm,
                slot_hbm, d0_hbm, d1_hbm, sc0_hbm, sc1_hbm,
                e1v, p0v, p1v, s0v, s1v, cntv, d0v, d1v, sc0v, sc1v,
                e0f, e1f, p0f, p1f, slotv):
    wid = lax.axis_index("s") * 2 + lax.axis_index("c")
    base = wid * TOKW
    pltpu.sync_copy(cnt_hbm, cntv)
    pltpu.sync_copy(e1_hbm.at[pl.ds(base, TOKW)], e1v)
    pltpu.sync_copy(p0_hbm.at[pl.ds(base, TOKW)], p0v)
    pltpu.sync_copy(p1_hbm.at[pl.ds(base, TOKW)], p1v)
    pltpu.sync_copy(s0_hbm.at[pl.ds(base, TOKW)], s0v)
    pltpu.sync_copy(s1_hbm.at[pl.ds(base, TOKW)], s1v)

    def chunk(c, _):
        sl = pl.ds(c * 16, 16)
        e1c = e1v[sl]
        p0c = p0v[sl]
        p1c = p1v[sl] + plsc.load_gather(cntv, [e1c])
        k0 = p0c < CAP
        k1 = p1c < CAP
        d0v[sl] = jnp.where(k0, p0c, 0)
        d1v[sl] = jnp.where(k1, e1c * CAP + p1c, 0)
        sc0v[sl] = s0v[sl] * k0.astype(jnp.float32)
        sc1v[sl] = s1v[sl] * k1.astype(jnp.float32)
        return 0

    lax.fori_loop(0, TOKW // 16, chunk, 0)

    def fix0(c, _):
        sl = pl.ds(c * 16, 16)
        d0v[sl] = d0v[sl] + e1v[sl] * CAP
        return 0

    pltpu.sync_copy(e0_hbm.at[pl.ds(base, TOKW)], e1v)
    lax.fori_loop(0, TOKW // 16, fix0, 0)
    pltpu.sync_copy(d0v, d0_hbm.at[pl.ds(base, TOKW)])
    pltpu.sync_copy(d1v, d1_hbm.at[pl.ds(base, TOKW)])
    pltpu.sync_copy(sc0v, sc0_hbm.at[pl.ds(base, TOKW)])
    pltpu.sync_copy(sc1v, sc1_hbm.at[pl.ds(base, TOKW)])

    @pl.when(wid == 0)
    def _build_slots():
        pltpu.sync_copy(e0_hbm, e0f)
        pltpu.sync_copy(e1_hbm, e1f)
        pltpu.sync_copy(p0_hbm, p0f)
        pltpu.sync_copy(p1_hbm, p1f)

        def zinit(j, _):
            slotv[pl.ds(j * 16, 16)] = jnp.zeros((16,), jnp.int32)
            return 0

        lax.fori_loop(0, SLOTS_PAD // 16, zinit, 0)

        def schunk(c, _):
            sl = pl.ds(c * 16, 16)
            tok = c * 16 + lax.iota(jnp.int32, 16)
            e0c = e0f[sl]
            e1c = e1f[sl]
            p0c = p0f[sl]
            p1c = p1f[sl] + plsc.load_gather(cntv, [e1c])
            d0 = jnp.where(p0c < CAP, e0c * CAP + p0c, TRASH)
            d1 = jnp.where(p1c < CAP, e1c * CAP + p1c, TRASH)
            plsc.store_scatter(slotv, [d0], tok)
            plsc.store_scatter(slotv, [d1], tok)
            return 0

        lax.fori_loop(0, T // 16, schunk, 0)
        pltpu.sync_copy(slotv, slot_hbm)


def _route(e0, e1, p0, p1r, s0, s1, cnt0):
    i32, f32 = jnp.int32, jnp.float32
    mesh = plsc.VectorSubcoreMesh(core_axis_name="c", subcore_axis_name="s")
    vm = pltpu.VMEM
    fn = pl.kernel(
        _route_body,
        out_type=(jax.ShapeDtypeStruct((SLOTS_PAD,), i32),
                  jax.ShapeDtypeStruct((T,), i32),
                  jax.ShapeDtypeStruct((T,), i32),
                  jax.ShapeDtypeStruct((T,), f32),
                  jax.ShapeDtypeStruct((T,), f32)),
        mesh=mesh,
        scratch_types=[vm((TOKW,), i32), vm((TOKW,), i32), vm((TOKW,), i32),
                       vm((TOKW,), f32), vm((TOKW,), f32),
                       vm((E,), i32),
                       vm((TOKW,), i32), vm((TOKW,), i32),
                       vm((TOKW,), f32), vm((TOKW,), f32),
                       vm((T,), i32), vm((T,), i32), vm((T,), i32),
                       vm((T,), i32), vm((SLOTS_PAD,), i32)],
        compiler_params=_SC_PARAMS,
    )
    return fn(e0, e1, p0, p1r, s0, s1, cnt0)


CHD = 40
NCHD = (SLOTS // NW) // CHD


def _dispatch_body(x_hbm, slot_hbm, disp_hbm, idxv, rowsv, sems):
    wid = lax.axis_index("s") * 2 + lax.axis_index("c")
    base = wid * (SLOTS // NW)

    def issue(c, b):
        off = base + c * CHD
        pltpu.sync_copy(slot_hbm.at[pl.ds(off, CHD)], idxv.at[b])
        pltpu.async_copy(x_hbm.at[idxv.at[b]], rowsv.at[b], sems.at[b])

    def flush(c, b):
        off = base + c * CHD
        pltpu.make_async_copy(x_hbm.at[pl.ds(0, CHD), :], rowsv.at[b],
                              sems.at[b]).wait()
        pltpu.sync_copy(rowsv.at[b], disp_hbm.at[pl.ds(off, CHD)])

    issue(0, 0)

    def pair(cc, _):
        c = cc * 2
        issue(c + 1, 1)
        flush(c, 0)

        @pl.when(cc < NCHD // 2 - 1)
        def _prefetch_next():
            issue(c + 2, 0)

        flush(c + 1, 1)
        return 0

    lax.fori_loop(0, NCHD // 2, pair, 0)


def _dispatch(x, slot):
    mesh = plsc.VectorSubcoreMesh(core_axis_name="c", subcore_axis_name="s")
    fn = pl.kernel(
        _dispatch_body,
        out_type=jax.ShapeDtypeStruct((SLOTS, D), jnp.float32),
        mesh=mesh,
        scratch_types=[pltpu.VMEM((2, CHD), jnp.int32),
                       pltpu.VMEM((2, CHD, D), jnp.float32),
                       pltpu.SemaphoreType.DMA((2,))],
        compiler_params=_SC_PARAMS,
    )
    return fn(x, slot)


def _ffn_body(xb_ref, w1_ref, b1_ref, w2_ref, b2_ref, y_ref):
    xb = xb_ref[0]
    h = jnp.dot(xb.astype(jnp.bfloat16), w1_ref[0].astype(jnp.bfloat16),
                preferred_element_type=jnp.float32)
    h = jax.nn.gelu(h + b1_ref[0])
    y = jnp.dot(h.astype(jnp.bfloat16), w2_ref[0].astype(jnp.bfloat16),
                preferred_element_type=jnp.float32)
    y_ref[0] = y + b2_ref[0]


def _ffn(disp, fc1_w, fc1_b, fc2_w, fc2_b):
    return pl.pallas_call(
        _ffn_body,
        grid=(E,),
        in_specs=[pl.BlockSpec((1, CAP, D), lambda e: (e, 0, 0)),
                  pl.BlockSpec((1, D, H), lambda e: (e, 0, 0)),
                  pl.BlockSpec((1, 1, H), lambda e: (e, 0, 0)),
                  pl.BlockSpec((1, H, D), lambda e: (e, 0, 0)),
                  pl.BlockSpec((1, 1, D), lambda e: (e, 0, 0))],
        out_specs=pl.BlockSpec((1, CAP, D), lambda e: (e, 0, 0)),
        out_shape=jax.ShapeDtypeStruct((E, CAP, D), jnp.float32),
    )(disp.reshape(E, CAP, D), fc1_w, fc1_b.reshape(E, 1, H),
      fc2_w, fc2_b.reshape(E, 1, D))


CHC = 32
NCH = TOKW // CHC


def _combine_body(y_hbm, d0_hbm, d1_hbm, sc0_hbm, sc1_hbm, out_hbm,
                  i0v, i1v, s0v, s1v, r0v, r1v, ov, sem0, sem1):
    wid = lax.axis_index("s") * 2 + lax.axis_index("c")
    base = wid * TOKW

    def chunk(c, _):
        off = base + c * CHC
        pltpu.sync_copy(d0_hbm.at[pl.ds(off, CHC)], i0v)
        pltpu.sync_copy(d1_hbm.at[pl.ds(off, CHC)], i1v)
        pltpu.sync_copy(sc0_hbm.at[pl.ds(off, CHC)], s0v)
        pltpu.sync_copy(sc1_hbm.at[pl.ds(off, CHC)], s1v)
        cp0 = pltpu.async_copy(y_hbm.at[i0v], r0v, sem0)
        cp1 = pltpu.async_copy(y_hbm.at[i1v], r1v, sem1)
        cp0.wait()
        cp1.wait()

        def tok(t, _):
            tvec = jnp.broadcast_to(t, (16,)).astype(jnp.int32)
            a = plsc.load_gather(s0v, [tvec])
            b = plsc.load_gather(s1v, [tvec])
            for k in range(D // 16):
                sl = pl.ds(k * 16, 16)
                ov[t, sl] = r0v[t, sl] * a + r1v[t, sl] * b
            return 0

        lax.fori_loop(0, CHC, tok, 0)
        pltpu.sync_copy(ov, out_hbm.at[pl.ds(off, CHC)])
        return 0

    lax.fori_loop(0, NCH, chunk, 0)


def _combine(y, d0, d1, sc0, sc1):
    mesh = plsc.VectorSubcoreMesh(core_axis_name="c", subcore_axis_name="s")
    vm = pltpu.VMEM
    fn = pl.kernel(
        _combine_body,
        out_type=jax.ShapeDtypeStruct((T, D), jnp.float32),
        mesh=mesh,
        scratch_types=[vm((CHC,), jnp.int32), vm((CHC,), jnp.int32),
                       vm((CHC,), jnp.float32), vm((CHC,), jnp.float32),
                       vm((CHC, D), jnp.float32), vm((CHC, D), jnp.float32),
                       vm((CHC, D), jnp.float32),
                       pltpu.SemaphoreType.DMA, pltpu.SemaphoreType.DMA],
        compiler_params=_SC_PARAMS,
    )
    return fn(y, d0, d1, sc0, sc1)


def kernel(x, wg, fc1_w, fc1_b, fc2_w, fc2_b):
    e0, e1, p0, p1r, s0, s1, stats, laux = _gate(x, wg)
    flat = lambda a: a.reshape(T)
    cnt0 = stats[0, :E].astype(jnp.int32)
    slot, d0, d1, sc0, sc1 = _route(flat(e0), flat(e1), flat(p0), flat(p1r),
                                    flat(s0), flat(s1), cnt0)
    disp = _dispatch(x, slot[:SLOTS])
    y = _ffn(disp, fc1_w, fc1_b, fc2_w, fc2_b)
    out = _combine(y.reshape(SLOTS, D), d0, d1, sc0, sc1)
    return out, laux.reshape(())


# speedup vs baseline: 1.1690x; 1.1690x over previous
import functools

import jax
import jax.numpy as jnp
from jax import lax
from jax.experimental import pallas as pl
from jax.experimental.pallas import tpu as pltpu
from jax.experimental.pallas import tpu_sc as plsc

E = 64
K = 2
D = 1024
H = 1024
T = 8192
CAP = 320
_SC_PARAMS = pltpu.CompilerParams(needs_layout_passes=False)
SLOTS = E * CAP
SLOTS_PAD = 20496
TRASH = SLOTS

NW = 32
TOKW = T // NW
TOKB = 512
NB = T // TOKB


def _gate_body(x_ref, wg_ref, e0_ref, e1_ref, p0_ref, p1_ref, s0_ref, s1_ref,
               stats_ref, laux_ref, acc_ref):
    i = pl.program_id(0)

    @pl.when(i == 0)
    def _init():
        acc_ref[...] = jnp.zeros_like(acc_ref)

    xb = x_ref[...]
    logits = jnp.dot(xb, wg_ref[...], preferred_element_type=jnp.float32)
    m = jnp.max(logits, axis=-1, keepdims=True)
    ex = jnp.exp(logits - m)
    gates = ex / jnp.sum(ex, axis=-1, keepdims=True)

    ii = lax.broadcasted_iota(jnp.int32, (TOKB, E), 1)
    m1 = jnp.max(gates, axis=-1, keepdims=True)
    idx1 = jnp.min(jnp.where(gates == m1, ii, E), axis=-1)
    mask0 = ii == idx1[:, None]
    g2 = jnp.where(mask0, -1.0, gates)
    m2 = jnp.max(g2, axis=-1, keepdims=True)
    idx2 = jnp.min(jnp.where(g2 == m2, ii, E), axis=-1)
    mask1 = ii == idx2[:, None]

    denom = m1[:, 0] + m2[:, 0] + 1e-9
    s0 = m1[:, 0] / denom
    s1 = m2[:, 0] / denom

    mask0f = mask0.astype(jnp.float32)
    mask1f = mask1.astype(jnp.float32)
    r = lax.broadcasted_iota(jnp.int32, (TOKB, TOKB), 0)
    c = lax.broadcasted_iota(jnp.int32, (TOKB, TOKB), 1)
    tril = (r >= c).astype(jnp.float32)
    cum0 = jnp.dot(tril, mask0f, preferred_element_type=jnp.float32)
    cum1 = jnp.dot(tril, mask1f, preferred_element_type=jnp.float32)

    carry0 = acc_ref[0:1, :E]
    carry1 = acc_ref[1:2, :E]
    p0 = jnp.sum(mask0f * (cum0 + carry0 - 1.0), axis=-1)
    p1 = jnp.sum(mask1f * (cum1 + carry1 - 1.0), axis=-1)

    e0_ref[...] = idx1.reshape(1, 1, TOKB)
    e1_ref[...] = idx2.reshape(1, 1, TOKB)
    p0_ref[...] = p0.astype(jnp.int32).reshape(1, 1, TOKB)
    p1_ref[...] = p1.astype(jnp.int32).reshape(1, 1, TOKB)
    s0_ref[...] = s0.reshape(1, 1, TOKB)
    s1_ref[...] = s1.reshape(1, 1, TOKB)

    acc_ref[0:1, :E] = carry0 + jnp.sum(mask0f, axis=0, keepdims=True)
    acc_ref[1:2, :E] = carry1 + jnp.sum(mask1f, axis=0, keepdims=True)
    acc_ref[2:3, :E] = acc_ref[2:3, :E] + jnp.sum(gates, axis=0, keepdims=True)

    stats_ref[...] = acc_ref[...]
    cnt0 = acc_ref[0:1, :E]
    me = acc_ref[2:3, :E]
    laux_ref[0, 0] = (jnp.float32(E) * jnp.sum(me * cnt0)
                      / (jnp.float32(T) * jnp.float32(T)))


def _gate(x, wg):
    i32, f32 = jnp.int32, jnp.float32
    tok3 = lambda dt: jax.ShapeDtypeStruct((NB, 1, TOKB), dt)
    out_shapes = (tok3(i32), tok3(i32), tok3(i32), tok3(i32),
                  tok3(f32), tok3(f32),
                  jax.ShapeDtypeStruct((8, 128), f32),
                  jax.ShapeDtypeStruct((1, 1), f32))
    tokspec = lambda: pl.BlockSpec((1, 1, TOKB), lambda i: (i, 0, 0))
    return pl.pallas_call(
        _gate_body,
        grid=(NB,),
        in_specs=[pl.BlockSpec((TOKB, D), lambda i: (i, 0)),
                  pl.BlockSpec((D, E), lambda i: (0, 0))],
        out_specs=(tokspec(), tokspec(), tokspec(), tokspec(),
                   tokspec(), tokspec(),
                   pl.BlockSpec((8, 128), lambda i: (0, 0)),
                   pl.BlockSpec(memory_space=pltpu.SMEM)),
        out_shape=out_shapes,
        scratch_shapes=[pltpu.VMEM((8, 128), f32)],
    )(x, wg)


def _route_body(e0_hbm, e1_hbm, p0_hbm, p1_hbm, s0_hbm, s1_hbm, cnt_hbm,
                slot_hbm, d0_hbm, d1_hbm, sc0_hbm, sc1_hbm,
                e1v, p0v, p1v, s0v, s1v, cntv, d0v, d1v, sc0v, sc1v,
                e0f, e1f, p0f, p1f, slotv):
    wid = lax.axis_index("s") * 2 + lax.axis_index("c")
    base = wid * TOKW
    pltpu.sync_copy(cnt_hbm, cntv)
    pltpu.sync_copy(e1_hbm.at[pl.ds(base, TOKW)], e1v)
    pltpu.sync_copy(p0_hbm.at[pl.ds(base, TOKW)], p0v)
    pltpu.sync_copy(p1_hbm.at[pl.ds(base, TOKW)], p1v)
    pltpu.sync_copy(s0_hbm.at[pl.ds(base, TOKW)], s0v)
    pltpu.sync_copy(s1_hbm.at[pl.ds(base, TOKW)], s1v)

    def chunk(c, _):
        sl = pl.ds(c * 16, 16)
        e1c = e1v[sl]
        p0c = p0v[sl]
        p1c = p1v[sl] + plsc.load_gather(cntv, [e1c])
        k0 = p0c < CAP
        k1 = p1c < CAP
        d0v[sl] = jnp.where(k0, p0c, 0)
        d1v[sl] = jnp.where(k1, e1c * CAP + p1c, 0)
        sc0v[sl] = s0v[sl] * k0.astype(jnp.float32)
        sc1v[sl] = s1v[sl] * k1.astype(jnp.float32)
        return 0

    lax.fori_loop(0, TOKW // 16, chunk, 0)

    def fix0(c, _):
        sl = pl.ds(c * 16, 16)
        d0v[sl] = d0v[sl] + e1v[sl] * CAP
        return 0

    pltpu.sync_copy(e0_hbm.at[pl.ds(base, TOKW)], e1v)
    lax.fori_loop(0, TOKW // 16, fix0, 0)
    pltpu.sync_copy(d0v, d0_hbm.at[pl.ds(base, TOKW)])
    pltpu.sync_copy(d1v, d1_hbm.at[pl.ds(base, TOKW)])
    pltpu.sync_copy(sc0v, sc0_hbm.at[pl.ds(base, TOKW)])
    pltpu.sync_copy(sc1v, sc1_hbm.at[pl.ds(base, TOKW)])

    @pl.when(wid == 0)
    def _build_slots():
        pltpu.sync_copy(e0_hbm, e0f)
        pltpu.sync_copy(e1_hbm, e1f)
        pltpu.sync_copy(p0_hbm, p0f)
        pltpu.sync_copy(p1_hbm, p1f)

        def zinit(j, _):
            slotv[pl.ds(j * 16, 16)] = jnp.zeros((16,), jnp.int32)
            return 0

        lax.fori_loop(0, SLOTS_PAD // 16, zinit, 0)

        def schunk(c, _):
            sl = pl.ds(c * 16, 16)
            tok = c * 16 + lax.iota(jnp.int32, 16)
            e0c = e0f[sl]
            e1c = e1f[sl]
            p0c = p0f[sl]
            p1c = p1f[sl] + plsc.load_gather(cntv, [e1c])
            d0 = jnp.where(p0c < CAP, e0c * CAP + p0c, TRASH)
            d1 = jnp.where(p1c < CAP, e1c * CAP + p1c, TRASH)
            plsc.store_scatter(slotv, [d0], tok)
            plsc.store_scatter(slotv, [d1], tok)
            return 0

        lax.fori_loop(0, T // 16, schunk, 0)
        pltpu.sync_copy(slotv, slot_hbm)


def _route(e0, e1, p0, p1r, s0, s1, cnt0):
    i32, f32 = jnp.int32, jnp.float32
    mesh = plsc.VectorSubcoreMesh(core_axis_name="c", subcore_axis_name="s")
    vm = pltpu.VMEM
    fn = pl.kernel(
        _route_body,
        out_type=(jax.ShapeDtypeStruct((SLOTS_PAD,), i32),
                  jax.ShapeDtypeStruct((T,), i32),
                  jax.ShapeDtypeStruct((T,), i32),
                  jax.ShapeDtypeStruct((T,), f32),
                  jax.ShapeDtypeStruct((T,), f32)),
        mesh=mesh,
        scratch_types=[vm((TOKW,), i32), vm((TOKW,), i32), vm((TOKW,), i32),
                       vm((TOKW,), f32), vm((TOKW,), f32),
                       vm((E,), i32),
                       vm((TOKW,), i32), vm((TOKW,), i32),
                       vm((TOKW,), f32), vm((TOKW,), f32),
                       vm((T,), i32), vm((T,), i32), vm((T,), i32),
                       vm((T,), i32), vm((SLOTS_PAD,), i32)],
        compiler_params=_SC_PARAMS,
    )
    return fn(e0, e1, p0, p1r, s0, s1, cnt0)


def _ffn_body(slot_ref, x_ref, w1_ref, b1_ref, w2_ref, b2_ref, y_ref,
              xb_ref, sems):
    g = pl.program_id(0)
    buf = lax.rem(g, 2)

    @pl.when(g < E)
    def _issue():
        def it(j, _):
            s = slot_ref[0, 0, j]
            pltpu.make_async_copy(
                x_ref.at[pl.ds(s, 1), :],
                xb_ref.at[buf, pl.ds(j, 1), :],
                sems.at[buf]).start()
            return 0

        lax.fori_loop(0, CAP, it, 0, unroll=8)

    @pl.when(g >= 1)
    def _compute():
        pbuf = lax.rem(g + 1, 2)
        pltpu.make_async_copy(x_ref.at[pl.ds(0, CAP), :], xb_ref.at[pbuf],
                              sems.at[pbuf]).wait()
        xb = xb_ref[pbuf]
        h = jnp.dot(xb.astype(jnp.bfloat16), w1_ref[0].astype(jnp.bfloat16),
                    preferred_element_type=jnp.float32)
        h = jax.nn.gelu(h + b1_ref[0])
        y = jnp.dot(h.astype(jnp.bfloat16), w2_ref[0].astype(jnp.bfloat16),
                    preferred_element_type=jnp.float32)
        y_ref[0] = y + b2_ref[0]


def _ffn(x, slot, fc1_w, fc1_b, fc2_w, fc2_b):
    prev = lambda g: jnp.maximum(g - 1, 0)
    return pl.pallas_call(
        _ffn_body,
        grid=(E + 1,),
        in_specs=[pl.BlockSpec((1, 1, CAP),
                               lambda g: (jnp.minimum(g, E - 1), 0, 0),
                               memory_space=pltpu.SMEM),
                  pl.BlockSpec(memory_space=pl.ANY),
                  pl.BlockSpec((1, D, H), lambda g: (prev(g), 0, 0)),
                  pl.BlockSpec((1, 1, H), lambda g: (prev(g), 0, 0)),
                  pl.BlockSpec((1, H, D), lambda g: (prev(g), 0, 0)),
                  pl.BlockSpec((1, 1, D), lambda g: (prev(g), 0, 0))],
        out_specs=pl.BlockSpec((1, CAP, D), lambda g: (prev(g), 0, 0)),
        out_shape=jax.ShapeDtypeStruct((E, CAP, D), jnp.float32),
        scratch_shapes=[pltpu.VMEM((2, CAP, D), jnp.float32),
                        pltpu.SemaphoreType.DMA((2,))],
    )(slot.reshape(E, 1, CAP), x, fc1_w, fc1_b.reshape(E, 1, H),
      fc2_w, fc2_b.reshape(E, 1, D))


CHC = 32
NCH = TOKW // CHC


def _combine_body(y_hbm, d0_hbm, d1_hbm, sc0_hbm, sc1_hbm, out_hbm,
                  i0v, i1v, s0v, s1v, r0v, r1v, ov, sem0, sem1):
    wid = lax.axis_index("s") * 2 + lax.axis_index("c")
    base = wid * TOKW

    def chunk(c, _):
        off = base + c * CHC
        pltpu.sync_copy(d0_hbm.at[pl.ds(off, CHC)], i0v)
        pltpu.sync_copy(d1_hbm.at[pl.ds(off, CHC)], i1v)
        pltpu.sync_copy(sc0_hbm.at[pl.ds(off, CHC)], s0v)
        pltpu.sync_copy(sc1_hbm.at[pl.ds(off, CHC)], s1v)
        cp0 = pltpu.async_copy(y_hbm.at[i0v], r0v, sem0)
        cp1 = pltpu.async_copy(y_hbm.at[i1v], r1v, sem1)
        cp0.wait()
        cp1.wait()

        def tok(t, _):
            tvec = jnp.broadcast_to(t, (16,)).astype(jnp.int32)
            a = plsc.load_gather(s0v, [tvec])
            b = plsc.load_gather(s1v, [tvec])
            for k in range(D // 16):
                sl = pl.ds(k * 16, 16)
                ov[t, sl] = r0v[t, sl] * a + r1v[t, sl] * b
            return 0

        lax.fori_loop(0, CHC, tok, 0)
        pltpu.sync_copy(ov, out_hbm.at[pl.ds(off, CHC)])
        return 0

    lax.fori_loop(0, NCH, chunk, 0)


def _combine(y, d0, d1, sc0, sc1):
    mesh = plsc.VectorSubcoreMesh(core_axis_name="c", subcore_axis_name="s")
    vm = pltpu.VMEM
    fn = pl.kernel(
        _combine_body,
        out_type=jax.ShapeDtypeStruct((T, D), jnp.float32),
        mesh=mesh,
        scratch_types=[vm((CHC,), jnp.int32), vm((CHC,), jnp.int32),
                       vm((CHC,), jnp.float32), vm((CHC,), jnp.float32),
                       vm((CHC, D), jnp.float32), vm((CHC, D), jnp.float32),
                       vm((CHC, D), jnp.float32),
                       pltpu.SemaphoreType.DMA, pltpu.SemaphoreType.DMA],
        compiler_params=_SC_PARAMS,
    )
    return fn(y, d0, d1, sc0, sc1)


def kernel(x, wg, fc1_w, fc1_b, fc2_w, fc2_b):
    e0, e1, p0, p1r, s0, s1, stats, laux = _gate(x, wg)
    flat = lambda a: a.reshape(T)
    cnt0 = stats[0, :E].astype(jnp.int32)
    slot, d0, d1, sc0, sc1 = _route(flat(e0), flat(e1), flat(p0), flat(p1r),
                                    flat(s0), flat(s1), cnt0)
    y = _ffn(x, slot[:SLOTS], fc1_w, fc1_b, fc2_w, fc2_b)
    out = _combine(y.reshape(SLOTS, D), d0, d1, sc0, sc1)
    return out, laux.reshape(())
